# Initial kernel scaffold; baseline (speedup 1.0000x reference)
import jax
import jax.numpy as jnp
from jax.experimental import pallas as pl


def kernel(drug_graph, drug_sim_feat, dis_graph, disease_sim_feat, W1_drug, b1_drug, W2_drug, b2_drug, W1_dis, b1_dis, W2_dis, b2_dis):
    raise NotImplementedError("write your pallas kernel here")


# speedup vs baseline: 1.0283x; 1.0283x over previous
import jax
import jax.numpy as jnp
from jax.experimental import pallas as pl
from jax.experimental.pallas import tpu as pltpu


def _layer1_kernel(adj_ref, x_ref, w1_ref, b1_ref, w2_ref, out_ref, s1_ref):
    @pl.when(pl.program_id(0) == 0)
    def _():
        s1_ref[...] = jnp.dot(x_ref[...], w1_ref[...],
                              preferred_element_type=jnp.float32)

    acc = jnp.dot(adj_ref[...], s1_ref[...],
                  preferred_element_type=jnp.float32)
    h = jnp.maximum(acc + b1_ref[...], 0.0)
    out_ref[...] = jnp.dot(h, w2_ref[...],
                           preferred_element_type=jnp.float32)


def _layer2_kernel(adj_ref, h2_ref, b2_ref, out_ref):
    acc = jnp.dot(adj_ref[...], h2_ref[...],
                  preferred_element_type=jnp.float32)
    out_ref[...] = acc + b2_ref[...]


def _pick_bm(n):
    for bm in (400, 200, 80, 40, 8):
        if n % bm == 0:
            return bm
    return min(n, 256)


def _gcn(adj, x, w1, b1, w2, b2):
    n, f = x.shape
    bm = _pick_bm(n)
    grid = (pl.cdiv(n, bm),)
    full = lambda r, c: pl.BlockSpec((r, c), lambda m: (0, 0))

    h2 = pl.pallas_call(
        _layer1_kernel,
        grid=grid,
        in_specs=[
            pl.BlockSpec((bm, n), lambda m: (m, 0)),
            full(n, f),
            full(f, f),
            full(1, f),
            full(f, f),
        ],
        out_specs=pl.BlockSpec((bm, f), lambda m: (m, 0)),
        out_shape=jax.ShapeDtypeStruct((n, f), jnp.float32),
        scratch_shapes=[pltpu.VMEM((n, f), jnp.float32)],
        compiler_params=pltpu.CompilerParams(
            dimension_semantics=("arbitrary",)),
    )(adj, x, w1, b1.reshape(1, f), w2)

    out = pl.pallas_call(
        _layer2_kernel,
        grid=grid,
        in_specs=[
            pl.BlockSpec((bm, n), lambda m: (m, 0)),
            full(n, f),
            full(1, f),
        ],
        out_specs=pl.BlockSpec((bm, f), lambda m: (m, 0)),
        out_shape=jax.ShapeDtypeStruct((n, f), jnp.float32),
        compiler_params=pltpu.CompilerParams(
            dimension_semantics=("arbitrary",)),
    )(adj, h2, b2.reshape(1, f))
    return out


def kernel(drug_graph, drug_sim_feat, dis_graph, disease_sim_feat,
           W1_drug, b1_drug, W2_drug, b2_drug,
           W1_dis, b1_dis, W2_dis, b2_dis):
    emb1 = _gcn(drug_graph, drug_sim_feat, W1_drug, b1_drug, W2_drug, b2_drug)
    emb2 = _gcn(dis_graph, disease_sim_feat, W1_dis, b1_dis, W2_dis, b2_dis)
    return (emb1, emb2, emb1, emb2)
